# baseline (speedup 1.0000x reference)
import functools

import jax
import jax.numpy as jnp
from jax import lax
from jax.experimental import pallas as pl
from jax.experimental.pallas import tpu as pltpu
from jax.experimental.pallas import tpu_sc as plsc

N = 16384
C = 1000
CPAD = 1024
BR = 1024
NCHUNK = N // BR
NW = 16
PH = 2
EPW = N // PH // NW
RROWS = CPAD // 16


def _losses_body(x0_ref, x1_ref, l0_ref, l1_ref, o0_ref, o1_ref):
    ones = jnp.ones((C, 1), jnp.float32)
    for x_ref, lab_ref, o_ref in ((x0_ref, l0_ref, o0_ref),
                                  (x1_ref, l1_ref, o1_ref)):
        x = x_ref[...]
        lab = lab_ref[...]
        m = jnp.max(x, axis=1)
        xm = x - m[:, None]
        e = jnp.exp(xm)
        col = lax.broadcasted_iota(jnp.int32, x.shape, 1)
        sel = jnp.where(col == lab[:, None], xm, 0.0)
        s = jnp.sum(e, axis=1)
        p = jnp.sum(sel, axis=1)
        o_ref[...] = jnp.log(s) - p


def _losses(inputs, labels, phase):
    cb = phase * (NCHUNK // PH)
    return pl.pallas_call(
        _losses_body,
        grid=(N // (PH * 2 * BR),),
        in_specs=[
            pl.BlockSpec((BR, C), lambda i: (cb + 2 * i, 0)),
            pl.BlockSpec((BR, C), lambda i: (cb + 2 * i + 1, 0)),
            pl.BlockSpec((BR,), lambda i: (cb + 2 * i,)),
            pl.BlockSpec((BR,), lambda i: (cb + 2 * i + 1,)),
        ],
        out_specs=[
            pl.BlockSpec((BR,), lambda i: (i,)),
            pl.BlockSpec((BR,), lambda i: (i,)),
        ],
        out_shape=[
            jax.ShapeDtypeStruct((N // PH // 2,), jnp.float32),
            jax.ShapeDtypeStruct((N // PH // 2,), jnp.float32),
        ],
    )(inputs, inputs, labels, labels)


def _groupby(loss_even, loss_odd, labels, phase):
    mesh = plsc.VectorSubcoreMesh(
        core_axis_name="c", subcore_axis_name="s", num_cores=1)

    @functools.partial(
        pl.kernel,
        mesh=mesh,
        compiler_params=pltpu.CompilerParams(
            use_tc_tiling_on_sc=False, needs_layout_passes=False),
        out_type=[
            jax.ShapeDtypeStruct((RROWS, 16), jnp.float32),
            jax.ShapeDtypeStruct((RROWS, 16), jnp.float32),
        ],
        scratch_types=[
            pltpu.VMEM((EPW,), jnp.int32),
            pltpu.VMEM((EPW,), jnp.float32),
            pltpu.VMEM((16 * CPAD,), jnp.float32),
            pltpu.VMEM((16 * CPAD,), jnp.float32),
            pltpu.VMEM((RROWS, 16), jnp.float32),
            pltpu.VMEM((RROWS, 16), jnp.float32),
            pltpu.VMEM((RROWS,), jnp.int32),
            pltpu.VMEM_SHARED((RROWS, 16), jnp.float32),
            pltpu.VMEM_SHARED((RROWS, 16), jnp.float32),
        ],
    )
    def k(le_hbm, lo_hbm, lab_hbm, sum_hbm, cnt_hbm,
          lab_v, loss_v, bins, cbins, red_s, red_c, idx_v, sh_s, sh_c):
        wid = lax.axis_index("s")
        lane = lax.broadcasted_iota(jnp.int32, (16,), 0)
        zeros = jnp.zeros((16,), jnp.float32)
        ones = jnp.ones((16,), jnp.float32)

        def zero_body(j, _):
            for u in range(16):
                bins[pl.ds(j * 256 + u * 16, 16)] = zeros
                cbins[pl.ds(j * 256 + u * 16, 16)] = zeros
            return 0
        lax.fori_loop(0, 16 * CPAD // 256, zero_body, 0)
        for j in range(RROWS):
            red_s[j, :] = zeros
            red_c[j, :] = zeros
        for j in range(RROWS // 16):
            idx_v[pl.ds(j * 16, 16)] = lane + (j * 16)

        @pl.when(wid == 0)
        def _():
            pltpu.sync_copy(red_s, sh_s)
            pltpu.sync_copy(red_c, sh_c)
        plsc.subcore_barrier()

        half = wid // 8
        r = wid - half * 8
        loss_base = r * EPW
        lab_base = (phase * (N // PH)
                    + (2 * (loss_base // BR) + half) * BR
                    + loss_base - (loss_base // BR) * BR)
        pltpu.sync_copy(lab_hbm.at[pl.ds(lab_base, EPW)], lab_v)

        @pl.when(half == 0)
        def _():
            pltpu.sync_copy(le_hbm.at[pl.ds(loss_base, EPW)], loss_v)

        @pl.when(half == 1)
        def _():
            pltpu.sync_copy(lo_hbm.at[pl.ds(loss_base, EPW)], loss_v)

        lane_base = lane * CPAD

        def acc_body(i, _):
            for u in range(8):
                lab = lab_v[pl.ds(i * 128 + u * 16, 16)]
                lv = loss_v[pl.ds(i * 128 + u * 16, 16)]
                idx = lane_base + lab
                plsc.addupdate_scatter(bins, [idx], lv)
                plsc.addupdate_scatter(cbins, [idx], ones)
            return 0
        lax.fori_loop(0, EPW // 128, acc_body, 0)

        for j in range(RROWS):
            acc_s = bins[pl.ds(j * 16, 16)]
            acc_c = cbins[pl.ds(j * 16, 16)]
            for l in range(1, 16):
                acc_s = acc_s + bins[pl.ds(l * CPAD + j * 16, 16)]
                acc_c = acc_c + cbins[pl.ds(l * CPAD + j * 16, 16)]
            red_s[j, :] = acc_s
            red_c[j, :] = acc_c

        pltpu.sync_copy(red_s, sh_s.at[idx_v], add=True)
        pltpu.sync_copy(red_c, sh_c.at[idx_v], add=True)
        plsc.subcore_barrier()

        @pl.when(wid == 0)
        def _():
            pltpu.sync_copy(sh_s, sum_hbm)
            pltpu.sync_copy(sh_c, cnt_hbm)

    return k(loss_even, loss_odd, labels)


def kernel(inputs, labels):
    labels = labels.astype(jnp.int32)
    le0, lo0 = _losses(inputs, labels, 0)
    le1, lo1 = _losses(inputs, labels, 1)
    s0, c0 = _groupby(le0, lo0, labels, 0)
    s1, c1 = _groupby(le1, lo1, labels, 1)
    return ((s0 + s1).reshape(-1)[:C], (c0 + c1).reshape(-1)[:C])


# speedup vs baseline: 1.1021x; 1.1021x over previous
import functools

import jax
import jax.numpy as jnp
from jax import lax
from jax.experimental import pallas as pl
from jax.experimental.pallas import tpu as pltpu
from jax.experimental.pallas import tpu_sc as plsc

N = 16384
C = 1000
CPAD = 1024
BR = 1024
NCHUNK = N // BR
NW = 16
EPW = N // NW
RROWS = CPAD // 16


def _losses_body(x0_ref, x1_ref, l0_ref, l1_ref, o0_ref, o1_ref):
    ones = jnp.ones((C, 1), jnp.float32)
    for x_ref, lab_ref, o_ref in ((x0_ref, l0_ref, o0_ref),
                                  (x1_ref, l1_ref, o1_ref)):
        x = x_ref[...]
        lab = lab_ref[...]
        m = jnp.max(x, axis=1)
        xm = x - m[:, None]
        e = jnp.exp(xm)
        col = lax.broadcasted_iota(jnp.int32, x.shape, 1)
        sel = jnp.where(col == lab[:, None], xm, 0.0)
        s = jnp.sum(e, axis=1)
        p = jnp.sum(sel, axis=1)
        o_ref[...] = jnp.log(s) - p


def _losses(inputs, labels):
    return pl.pallas_call(
        _losses_body,
        grid=(N // (2 * BR),),
        in_specs=[
            pl.BlockSpec((BR, C), lambda i: (2 * i, 0)),
            pl.BlockSpec((BR, C), lambda i: (2 * i + 1, 0)),
            pl.BlockSpec((BR,), lambda i: (2 * i,)),
            pl.BlockSpec((BR,), lambda i: (2 * i + 1,)),
        ],
        out_specs=[
            pl.BlockSpec((BR,), lambda i: (i,)),
            pl.BlockSpec((BR,), lambda i: (i,)),
        ],
        out_shape=[
            jax.ShapeDtypeStruct((N // 2,), jnp.float32),
            jax.ShapeDtypeStruct((N // 2,), jnp.float32),
        ],
    )(inputs, inputs, labels, labels)


def _groupby(loss_even, loss_odd, labels):
    mesh = plsc.VectorSubcoreMesh(
        core_axis_name="c", subcore_axis_name="s", num_cores=1)

    @functools.partial(
        pl.kernel,
        mesh=mesh,
        compiler_params=pltpu.CompilerParams(
            use_tc_tiling_on_sc=False, needs_layout_passes=False),
        out_type=[
            jax.ShapeDtypeStruct((RROWS, 16), jnp.float32),
            jax.ShapeDtypeStruct((RROWS, 16), jnp.float32),
        ],
        scratch_types=[
            pltpu.VMEM((EPW,), jnp.int32),
            pltpu.VMEM((EPW,), jnp.float32),
            pltpu.VMEM((16 * CPAD,), jnp.float32),
            pltpu.VMEM((16 * CPAD,), jnp.float32),
            pltpu.VMEM((RROWS, 16), jnp.float32),
            pltpu.VMEM((RROWS, 16), jnp.float32),
            pltpu.VMEM((RROWS,), jnp.int32),
            pltpu.VMEM_SHARED((RROWS, 16), jnp.float32),
            pltpu.VMEM_SHARED((RROWS, 16), jnp.float32),
        ],
    )
    def k(le_hbm, lo_hbm, lab_hbm, sum_hbm, cnt_hbm,
          lab_v, loss_v, bins, cbins, red_s, red_c, idx_v, sh_s, sh_c):
        wid = lax.axis_index("s")
        lane = lax.broadcasted_iota(jnp.int32, (16,), 0)
        zeros = jnp.zeros((16,), jnp.float32)
        ones = jnp.ones((16,), jnp.float32)

        def zero_body(j, _):
            for u in range(16):
                bins[pl.ds(j * 256 + u * 16, 16)] = zeros
                cbins[pl.ds(j * 256 + u * 16, 16)] = zeros
            return 0
        lax.fori_loop(0, 16 * CPAD // 256, zero_body, 0)
        for j in range(RROWS):
            red_s[j, :] = zeros
            red_c[j, :] = zeros
        for j in range(RROWS // 16):
            idx_v[pl.ds(j * 16, 16)] = lane + (j * 16)

        @pl.when(wid == 0)
        def _():
            pltpu.sync_copy(red_s, sh_s)
            pltpu.sync_copy(red_c, sh_c)
        plsc.subcore_barrier()

        half = wid // 8
        r = wid - half * 8
        loss_base = r * EPW
        lab_base = ((2 * (loss_base // BR) + half) * BR
                    + loss_base - (loss_base // BR) * BR)
        pltpu.sync_copy(lab_hbm.at[pl.ds(lab_base, EPW)], lab_v)

        @pl.when(half == 0)
        def _():
            pltpu.sync_copy(le_hbm.at[pl.ds(loss_base, EPW)], loss_v)

        @pl.when(half == 1)
        def _():
            pltpu.sync_copy(lo_hbm.at[pl.ds(loss_base, EPW)], loss_v)

        lane_base = lane * CPAD

        def acc_body(i, _):
            for u in range(8):
                lab = lab_v[pl.ds(i * 128 + u * 16, 16)]
                lv = loss_v[pl.ds(i * 128 + u * 16, 16)]
                idx = lane_base + lab
                plsc.addupdate_scatter(bins, [idx], lv)
                plsc.addupdate_scatter(cbins, [idx], ones)
            return 0
        lax.fori_loop(0, EPW // 128, acc_body, 0)

        for j in range(RROWS):
            acc_s = bins[pl.ds(j * 16, 16)]
            acc_c = cbins[pl.ds(j * 16, 16)]
            for l in range(1, 16):
                acc_s = acc_s + bins[pl.ds(l * CPAD + j * 16, 16)]
                acc_c = acc_c + cbins[pl.ds(l * CPAD + j * 16, 16)]
            red_s[j, :] = acc_s
            red_c[j, :] = acc_c

        pltpu.sync_copy(red_s, sh_s.at[idx_v], add=True)
        pltpu.sync_copy(red_c, sh_c.at[idx_v], add=True)
        plsc.subcore_barrier()

        @pl.when(wid == 0)
        def _():
            pltpu.sync_copy(sh_s, sum_hbm)
            pltpu.sync_copy(sh_c, cnt_hbm)

    return k(loss_even, loss_odd, labels)


def kernel(inputs, labels):
    labels = labels.astype(jnp.int32)
    loss_even, loss_odd = _losses(inputs, labels)
    sums, counts = _groupby(loss_even, loss_odd, labels)
    return sums.reshape(-1)[:C], counts.reshape(-1)[:C]
